# baseline (speedup 1.0000x reference)
import jax
import jax.numpy as jnp
from jax.experimental import pallas as pl


_D = 65536
_d = 1024
_BLK = 4096


def _matvec_body(p_ref, z_ref, t_ref, o_ref):
    acc = jnp.sum(p_ref[...] * z_ref[...], axis=1)
    o_ref[...] = t_ref[...] + acc


def kernel(z, P, theta_base):
    D, d = P.shape
    zb = z.reshape(1, d)
    out = pl.pallas_call(
        _matvec_body,
        grid=(D // _BLK,),
        in_specs=[
            pl.BlockSpec((_BLK, d), lambda i: (i, 0)),
            pl.BlockSpec((1, d), lambda i: (0, 0)),
            pl.BlockSpec((_BLK,), lambda i: (i,)),
        ],
        out_specs=pl.BlockSpec((_BLK,), lambda i: (i,)),
        out_shape=jax.ShapeDtypeStruct((D,), jnp.float32),
    )(P, zb, theta_base)
    return out
